# baseline (speedup 1.0000x reference)
import jax
import jax.numpy as jnp
from jax import lax
from jax.experimental import pallas as pl
from jax.experimental.pallas import tpu as pltpu

V = 100000
E = 16
B = 1024

TILE_V = 2048
GRID_V = (V + TILE_V - 1) // TILE_V


def _gather_kernel(idx_ref, table_ref, emb_ref):
    def body(i, _):
        v = idx_ref[i]
        emb_ref[pl.ds(i, 1), :] = table_ref[pl.ds(v, 1), :]
        return 0

    lax.fori_loop(0, B, body, 0)


def _tc_gather(idx, table):
    return pl.pallas_call(
        _gather_kernel,
        in_specs=[
            pl.BlockSpec(memory_space=pltpu.SMEM),
            pl.BlockSpec((V, E), lambda: (0, 0)),
        ],
        out_specs=pl.BlockSpec((B, E), lambda: (0, 0)),
        out_shape=jax.ShapeDtypeStruct((B, E), jnp.float32),
    )(idx, table)


def _proj_kernel(emb_ref, w_ref, b_ref, out_ref):
    acc = lax.dot_general(
        emb_ref[...],
        w_ref[...],
        dimension_numbers=(((1,), (1,)), ((), ())),
        preferred_element_type=jnp.float32,
    )
    out_ref[0, :, :] = acc + b_ref[...][None, :]


def _tc_project(emb, w, b):
    return pl.pallas_call(
        _proj_kernel,
        grid=(GRID_V,),
        in_specs=[
            pl.BlockSpec((B, E), lambda j: (0, 0)),
            pl.BlockSpec((TILE_V, E), lambda j: (j, 0)),
            pl.BlockSpec((TILE_V,), lambda j: (j,)),
        ],
        out_specs=pl.BlockSpec((1, B, TILE_V), lambda j: (0, 0, j)),
        out_shape=jax.ShapeDtypeStruct((1, B, V), jnp.float32),
    )(emb, w, b)


def kernel(target, emb_table, W0, b0):
    idx = target.astype(jnp.int32)
    emb = _tc_gather(idx, emb_table)
    out = _tc_project(emb, W0, b0)
    return jnp.swapaxes(out, 0, 1)


# speedup vs baseline: 1.8469x; 1.8469x over previous
import jax
import jax.numpy as jnp
from jax import lax
from jax.experimental import pallas as pl
from jax.experimental.pallas import tpu as pltpu

V = 100000
E = 16
B = 1024

TILE_V = 2048
GRID_V = (V + TILE_V - 1) // TILE_V


def _gather_kernel(idx_ref, table_ref, emb_ref):
    def body(i, _):
        v = idx_ref[i]
        emb_ref[pl.ds(i, 1), :] = table_ref[pl.ds(v, 1), :]
        return 0

    lax.fori_loop(0, B, body, 0)


def _tc_gather(idx, table):
    return pl.pallas_call(
        _gather_kernel,
        in_specs=[
            pl.BlockSpec(memory_space=pltpu.SMEM),
            pl.BlockSpec((V, E), lambda: (0, 0)),
        ],
        out_specs=pl.BlockSpec((B, E), lambda: (0, 0)),
        out_shape=jax.ShapeDtypeStruct((B, E), jnp.float32),
    )(idx, table)


def _proj_kernel(w_ref, emb_ref, b_ref, out_ref):
    acc = lax.dot_general(
        w_ref[...],
        emb_ref[...],
        dimension_numbers=(((1,), (1,)), ((), ())),
        preferred_element_type=jnp.float32,
    )
    out_ref[...] = acc + b_ref[...]


def _tc_project(w, emb, b):
    return pl.pallas_call(
        _proj_kernel,
        grid=(GRID_V,),
        in_specs=[
            pl.BlockSpec((TILE_V, E), lambda j: (j, 0)),
            pl.BlockSpec((B, E), lambda j: (0, 0)),
            pl.BlockSpec((TILE_V, 1), lambda j: (j, 0)),
        ],
        out_specs=pl.BlockSpec((TILE_V, B), lambda j: (j, 0)),
        out_shape=jax.ShapeDtypeStruct((V, B), jnp.float32),
    )(w, emb, b)


def kernel(target, emb_table, W0, b0):
    idx = target.astype(jnp.int32)
    emb = _tc_gather(idx, emb_table)
    out_t = _tc_project(W0, emb, b0.reshape(V, 1))
    return jnp.swapaxes(out_t, 0, 1)[:, None, :]


# speedup vs baseline: 2.1734x; 1.1768x over previous
import jax
import jax.numpy as jnp
from jax import lax
from jax.experimental import pallas as pl
from jax.experimental.pallas import tpu as pltpu

V = 100000
E = 16
B = 1024

TILE_V = 2048
GRID_V = (V + TILE_V - 1) // TILE_V


def _gather_kernel(idx_ref, table_ref, emb_ref):
    def body(i, _):
        v = idx_ref[i]
        emb_ref[pl.ds(i, 1), :] = table_ref[pl.ds(v, 1), :]
        return 0

    lax.fori_loop(0, B, body, 0)


def _tc_gather(idx, table):
    return pl.pallas_call(
        _gather_kernel,
        in_specs=[
            pl.BlockSpec(memory_space=pltpu.SMEM),
            pl.BlockSpec((V, E), lambda: (0, 0)),
        ],
        out_specs=pl.BlockSpec((B, E), lambda: (0, 0)),
        out_shape=jax.ShapeDtypeStruct((B, E), jnp.float32),
    )(idx, table)


def _proj_kernel(w_ref, emb_ref, b_ref, out_ref):
    acc = lax.dot_general(
        w_ref[...],
        emb_ref[...],
        dimension_numbers=(((0,), (1,)), ((), ())),
        preferred_element_type=jnp.float32,
    )
    out_ref[...] = acc + b_ref[...]


def _tc_project(w_t, emb, b):
    return pl.pallas_call(
        _proj_kernel,
        grid=(GRID_V,),
        in_specs=[
            pl.BlockSpec((E, TILE_V), lambda j: (0, j)),
            pl.BlockSpec((B, E), lambda j: (0, 0)),
            pl.BlockSpec((TILE_V, 1), lambda j: (j, 0)),
        ],
        out_specs=pl.BlockSpec((TILE_V, B), lambda j: (j, 0)),
        out_shape=jax.ShapeDtypeStruct((V, B), jnp.float32),
    )(w_t, emb, b)


def kernel(target, emb_table, W0, b0):
    idx = target.astype(jnp.int32)
    emb = _tc_gather(idx, emb_table)
    out_t = _tc_project(W0.T, emb, b0.reshape(V, 1))
    return jnp.swapaxes(out_t, 0, 1)[:, None, :]


# speedup vs baseline: 2.2052x; 1.0146x over previous
import jax
import jax.numpy as jnp
from jax import lax
from jax.experimental import pallas as pl
from jax.experimental.pallas import tpu as pltpu

V = 100000
E = 16
B = 1024

TILE_V = 4096
GRID_V = (V + TILE_V - 1) // TILE_V


def _gather_kernel(idx_ref, table_ref, emb_ref):
    def body(i, _):
        v = idx_ref[i]
        emb_ref[pl.ds(i, 1), :] = table_ref[pl.ds(v, 1), :]
        return 0

    lax.fori_loop(0, B, body, 0)


def _tc_gather(idx, table):
    return pl.pallas_call(
        _gather_kernel,
        in_specs=[
            pl.BlockSpec(memory_space=pltpu.SMEM),
            pl.BlockSpec((V, E), lambda: (0, 0)),
        ],
        out_specs=pl.BlockSpec((B, E), lambda: (0, 0)),
        out_shape=jax.ShapeDtypeStruct((B, E), jnp.float32),
    )(idx, table)


def _proj_kernel(w_ref, emb_ref, b_ref, out_ref):
    acc = lax.dot_general(
        w_ref[...],
        emb_ref[...],
        dimension_numbers=(((0,), (1,)), ((), ())),
        preferred_element_type=jnp.float32,
    )
    out_ref[...] = acc + b_ref[...]


def _tc_project(w_t, emb, b):
    return pl.pallas_call(
        _proj_kernel,
        grid=(GRID_V,),
        in_specs=[
            pl.BlockSpec((E, TILE_V), lambda j: (0, j)),
            pl.BlockSpec((B, E), lambda j: (0, 0)),
            pl.BlockSpec((TILE_V, 1), lambda j: (j, 0)),
        ],
        out_specs=pl.BlockSpec((TILE_V, B), lambda j: (j, 0)),
        out_shape=jax.ShapeDtypeStruct((V, B), jnp.float32),
    )(w_t, emb, b)


def kernel(target, emb_table, W0, b0):
    idx = target.astype(jnp.int32)
    emb = _tc_gather(idx, emb_table)
    out_t = _tc_project(W0.T, emb, b0.reshape(V, 1))
    return jnp.swapaxes(out_t, 0, 1)[:, None, :]


# speedup vs baseline: 2.2811x; 1.0344x over previous
import jax
import jax.numpy as jnp
from jax import lax
from jax.experimental import pallas as pl
from jax.experimental.pallas import tpu as pltpu

V = 100000
E = 16
B = 1024

TILE_V = 4096
GRID_V = (V + TILE_V - 1) // TILE_V


def _gather_kernel(idx_ref, table_ref, emb_ref, sem):
    def issue(i, _):
        v = idx_ref[i]
        pltpu.make_async_copy(
            table_ref.at[pl.ds(v, 1), :], emb_ref.at[pl.ds(i, 1), :], sem
        ).start()
        return 0

    lax.fori_loop(0, B, issue, 0)

    def drain(i, _):
        pltpu.make_async_copy(
            table_ref.at[pl.ds(0, 1), :], emb_ref.at[pl.ds(i, 1), :], sem
        ).wait()
        return 0

    lax.fori_loop(0, B, drain, 0)


def _tc_gather(idx, table):
    return pl.pallas_call(
        _gather_kernel,
        in_specs=[
            pl.BlockSpec(memory_space=pltpu.SMEM),
            pl.BlockSpec(memory_space=pl.ANY),
        ],
        out_specs=pl.BlockSpec((B, E), lambda: (0, 0)),
        out_shape=jax.ShapeDtypeStruct((B, E), jnp.float32),
        scratch_shapes=[pltpu.SemaphoreType.DMA],
    )(idx, table)


def _proj_kernel(w_ref, emb_ref, b_ref, out_ref):
    acc = lax.dot_general(
        w_ref[...],
        emb_ref[...],
        dimension_numbers=(((0,), (1,)), ((), ())),
        preferred_element_type=jnp.float32,
    )
    out_ref[...] = acc + b_ref[...]


def _tc_project(w_t, emb, b):
    return pl.pallas_call(
        _proj_kernel,
        grid=(GRID_V,),
        in_specs=[
            pl.BlockSpec((E, TILE_V), lambda j: (0, j)),
            pl.BlockSpec((B, E), lambda j: (0, 0)),
            pl.BlockSpec((TILE_V, 1), lambda j: (j, 0)),
        ],
        out_specs=pl.BlockSpec((TILE_V, B), lambda j: (j, 0)),
        out_shape=jax.ShapeDtypeStruct((V, B), jnp.float32),
    )(w_t, emb, b)


def kernel(target, emb_table, W0, b0):
    idx = target.astype(jnp.int32)
    emb = _tc_gather(idx, emb_table)
    out_t = _tc_project(W0.T, emb, b0.reshape(V, 1))
    return jnp.swapaxes(out_t, 0, 1)[:, None, :]
